# baseline (device time: 23674 ns/iter reference)
import jax
import jax.numpy as jnp
from jax import lax
from jax.experimental import pallas as pl
from jax.experimental.pallas import tpu as pltpu

N_DEV = 4
CW = 256


def kernel(dy, W):
    m, k = dy.shape
    d = W.shape[0]

    def body(dy_hbm, w_hbm, out_ref, dy_v, w_v, comm1, comm2,
             copy_sems, send_sems, recv_sems):
        p = lax.axis_index("i")
        a = jnp.bitwise_xor(p, 1)
        b = 3 - p

        barrier_sem = pltpu.get_barrier_semaphore()
        for nbr in [a, b]:
            pl.semaphore_signal(
                barrier_sem, inc=1,
                device_id=(nbr,), device_id_type=pl.DeviceIdType.MESH,
            )

        cp_dy = pltpu.make_async_copy(dy_hbm, dy_v, copy_sems.at[0])
        cp_w = pltpu.make_async_copy(w_hbm, w_v, copy_sems.at[1])
        cp_dy.start()
        cp_w.start()

        p_lt2 = p < 2
        p_even = (p % 2) == 0
        p_03 = (p == 0) | (p == 3)

        k1 = jnp.where(p_03, 0, 128)
        o2 = jnp.where(p_lt2, 0, 64)
        kB = jnp.where(p_lt2, 0, 128)
        oB = jnp.where(p_even, 0, 64)

        cp_dy.wait()
        cp_w.wait()
        out_ref[...] = lax.dot_general(
            dy_v[...], w_v[...],
            dimension_numbers=(((1,), (1,)), ((), ())),
            preferred_element_type=jnp.float32,
        )

        pl.semaphore_wait(barrier_sem, 2)

        def exch(idx, partner, row_off, nrows, dst):
            r = pltpu.make_async_remote_copy(
                src_ref=out_ref.at[pl.ds(row_off, nrows), :],
                dst_ref=dst,
                send_sem=send_sems.at[idx],
                recv_sem=recv_sems.at[idx],
                device_id=(partner,),
                device_id_type=pl.DeviceIdType.MESH,
            )
            r.start()
            return r

        r0 = exch(0, a, 128 - k1, 128, comm1.at[0])
        r1 = exch(1, b, 384 - kB, 128, comm1.at[1])

        r0.wait_recv()
        out_ref[pl.ds(k1 + 64 - o2, 64), :] += comm1[0, pl.ds(64 - o2, 64), :]
        r2 = exch(2, b, k1 + 64 - o2, 64, comm2.at[0])
        r1.wait_recv()
        out_ref[pl.ds(256 + kB + 64 - oB, 64), :] += comm1[1, pl.ds(64 - oB, 64), :]
        r3 = exch(3, a, 256 + kB + 64 - oB, 64, comm2.at[1])

        out_ref[pl.ds(k1 + o2, 64), :] += comm1[0, pl.ds(o2, 64), :]
        out_ref[pl.ds(256 + kB + oB, 64), :] += comm1[1, pl.ds(oB, 64), :]

        r2.wait_recv()
        out_ref[pl.ds(k1 + o2, 64), :] += comm2[0]
        r4 = exch(4, b, k1 + o2, 64, out_ref.at[pl.ds(k1 + o2, 64), :])
        r6 = exch(6, a, k1 + o2, 64, out_ref.at[pl.ds(k1 + o2, 64), :])
        r3.wait_recv()
        out_ref[pl.ds(256 + kB + oB, 64), :] += comm2[1]
        r5 = exch(5, a, 256 + kB + oB, 64,
                  out_ref.at[pl.ds(256 + kB + oB, 64), :])
        r7 = exch(7, b, 256 + kB + oB, 64,
                  out_ref.at[pl.ds(256 + kB + oB, 64), :])

        r4.wait_recv()
        r8 = exch(8, a, k1 + 64 - o2, 64,
                  out_ref.at[pl.ds(k1 + 64 - o2, 64), :])
        r5.wait_recv()
        r9 = exch(9, b, 256 + kB + 64 - oB, 64,
                  out_ref.at[pl.ds(256 + kB + 64 - oB, 64), :])

        r6.wait_recv()
        r7.wait_recv()
        r8.wait_recv()
        r9.wait_recv()
        for r in (r0, r1, r2, r3, r4, r5, r6, r7, r8, r9):
            r.wait_send()

    return pl.pallas_call(
        body,
        out_shape=jax.ShapeDtypeStruct((m, d), jnp.float32),
        in_specs=[
            pl.BlockSpec(memory_space=pltpu.MemorySpace.HBM),
            pl.BlockSpec(memory_space=pltpu.MemorySpace.HBM),
        ],
        out_specs=pl.BlockSpec(memory_space=pltpu.VMEM),
        scratch_shapes=[
            pltpu.VMEM((m, k), jnp.float32),
            pltpu.VMEM((d, k), jnp.float32),
            pltpu.VMEM((2, 128, d), jnp.float32),
            pltpu.VMEM((2, 64, d), jnp.float32),
            pltpu.SemaphoreType.DMA((2,)),
            pltpu.SemaphoreType.DMA((10,)),
            pltpu.SemaphoreType.DMA((10,)),
        ],
        compiler_params=pltpu.CompilerParams(collective_id=0),
    )(dy, W)


# device time: 21549 ns/iter; 1.0986x vs baseline; 1.0986x over previous
import jax
import jax.numpy as jnp
from jax import lax
from jax.experimental import pallas as pl
from jax.experimental.pallas import tpu as pltpu

N_DEV = 4
CW = 128
NC = 512 // CW
COLS = [(c * CW, c) for c in range(NC)]


def kernel(dy, W):
    m, k = dy.shape
    d = W.shape[0]

    def body(dy_hbm, w_hbm, out_ref, dy_v, w_v, comm1, comm2,
             copy_sems, send_sems, recv_sems):
        p = lax.axis_index("i")
        a = jnp.bitwise_xor(p, 1)
        b = 3 - p

        barrier_sem = pltpu.get_barrier_semaphore()
        for nbr in [a, b]:
            pl.semaphore_signal(
                barrier_sem, inc=1,
                device_id=(nbr,), device_id_type=pl.DeviceIdType.MESH,
            )

        cp_dy = pltpu.make_async_copy(dy_hbm, dy_v, copy_sems.at[0])
        cp_w = pltpu.make_async_copy(w_hbm, w_v, copy_sems.at[1])
        cp_dy.start()
        cp_w.start()

        p_lt2 = p < 2
        p_even = (p % 2) == 0
        p_03 = (p == 0) | (p == 3)

        k1 = jnp.where(p_03, 0, 128)
        o2 = jnp.where(p_lt2, 0, 64)
        kB = jnp.where(p_lt2, 0, 128)
        oB = jnp.where(p_even, 0, 64)

        cp_dy.wait()
        cp_w.wait()
        out_ref[...] = lax.dot_general(
            dy_v[...], w_v[...],
            dimension_numbers=(((1,), (1,)), ((), ())),
            preferred_element_type=jnp.float32,
        )

        pl.semaphore_wait(barrier_sem, 2)

        def sem_idx(step, bf, ci):
            return (step * 2 + bf) * NC + ci

        def exch(step, bf, partner, row_off, nrows, c_off, ci, dst):
            r = pltpu.make_async_remote_copy(
                src_ref=out_ref.at[pl.ds(row_off, nrows), pl.ds(c_off, CW)],
                dst_ref=dst,
                send_sem=send_sems.at[sem_idx(step, bf, ci)],
                recv_sem=recv_sems.at[sem_idx(step, bf, ci)],
                device_id=(partner,),
                device_id_type=pl.DeviceIdType.MESH,
            )
            r.start()
            return r

        s1 = {}
        for c, ci in COLS:
            s1["B1", ci] = exch(0, 0, a, 128 - k1, 128, c, ci,
                                comm1.at[0, :, pl.ds(c, CW)])
            s1["B2", ci] = exch(0, 1, b, 384 - kB, 128, c, ci,
                                comm1.at[1, :, pl.ds(c, CW)])

        s2 = {}
        for c, ci in COLS:
            s1["B1", ci].wait_recv()
            out_ref[pl.ds(k1 + 64 - o2, 64), pl.ds(c, CW)] += (
                comm1[0, pl.ds(64 - o2, 64), pl.ds(c, CW)])
            s2["B1", ci] = exch(1, 0, b, k1 + 64 - o2, 64, c, ci,
                                comm2.at[0, :, pl.ds(c, CW)])
            s1["B2", ci].wait_recv()
            out_ref[pl.ds(256 + kB + 64 - oB, 64), pl.ds(c, CW)] += (
                comm1[1, pl.ds(64 - oB, 64), pl.ds(c, CW)])
            s2["B2", ci] = exch(1, 1, a, 256 + kB + 64 - oB, 64, c, ci,
                                comm2.at[1, :, pl.ds(c, CW)])

        out_ref[pl.ds(k1 + o2, 64), :] += comm1[0, pl.ds(o2, 64), :]
        out_ref[pl.ds(256 + kB + oB, 64), :] += comm1[1, pl.ds(oB, 64), :]

        s3 = {}
        s4a = {}
        for c, ci in COLS:
            s2["B1", ci].wait_recv()
            out_ref[pl.ds(k1 + o2, 64), pl.ds(c, CW)] += (
                comm2[0, :, pl.ds(c, CW)])
            s3["B1", ci] = exch(2, 0, b, k1 + o2, 64, c, ci,
                                out_ref.at[pl.ds(k1 + o2, 64), pl.ds(c, CW)])
            s4a["B1", ci] = exch(3, 0, a, k1 + o2, 64, c, ci,
                                 out_ref.at[pl.ds(k1 + o2, 64), pl.ds(c, CW)])
            s2["B2", ci].wait_recv()
            out_ref[pl.ds(256 + kB + oB, 64), pl.ds(c, CW)] += (
                comm2[1, :, pl.ds(c, CW)])
            s3["B2", ci] = exch(
                2, 1, a, 256 + kB + oB, 64, c, ci,
                out_ref.at[pl.ds(256 + kB + oB, 64), pl.ds(c, CW)])
            s4a["B2", ci] = exch(
                3, 1, b, 256 + kB + oB, 64, c, ci,
                out_ref.at[pl.ds(256 + kB + oB, 64), pl.ds(c, CW)])

        s4b = {}
        for c, ci in COLS:
            s3["B1", ci].wait_recv()
            s4b["B1", ci] = exch(
                4, 0, a, k1 + 64 - o2, 64, c, ci,
                out_ref.at[pl.ds(k1 + 64 - o2, 64), pl.ds(c, CW)])
            s3["B2", ci].wait_recv()
            s4b["B2", ci] = exch(
                4, 1, b, 256 + kB + 64 - oB, 64, c, ci,
                out_ref.at[pl.ds(256 + kB + 64 - oB, 64), pl.ds(c, CW)])

        for grp in (s4a, s4b):
            for r in grp.values():
                r.wait_recv()
        for grp in (s1, s2, s3, s4a, s4b):
            for r in grp.values():
                r.wait_send()

    n_sems = 5 * 2 * NC
    return pl.pallas_call(
        body,
        out_shape=jax.ShapeDtypeStruct((m, d), jnp.float32),
        in_specs=[
            pl.BlockSpec(memory_space=pltpu.MemorySpace.HBM),
            pl.BlockSpec(memory_space=pltpu.MemorySpace.HBM),
        ],
        out_specs=pl.BlockSpec(memory_space=pltpu.VMEM),
        scratch_shapes=[
            pltpu.VMEM((m, k), jnp.float32),
            pltpu.VMEM((d, k), jnp.float32),
            pltpu.VMEM((2, 128, d), jnp.float32),
            pltpu.VMEM((2, 64, d), jnp.float32),
            pltpu.SemaphoreType.DMA((2,)),
            pltpu.SemaphoreType.DMA((n_sems,)),
            pltpu.SemaphoreType.DMA((n_sems,)),
        ],
        compiler_params=pltpu.CompilerParams(collective_id=0),
    )(dy, W)


# device time: 21408 ns/iter; 1.1058x vs baseline; 1.0066x over previous
import jax
import jax.numpy as jnp
from jax import lax
from jax.experimental import pallas as pl
from jax.experimental.pallas import tpu as pltpu

N_DEV = 4
CW = 128
NC = 512 // CW
COLS = [(c * CW, c) for c in range(NC)]


def kernel(dy, W):
    m, k = dy.shape
    d = W.shape[0]

    def body(dy_hbm, w_hbm, out_ref, dy_v, w_v, comm1, comm2,
             copy_sems, send_sems, recv_sems):
        p = lax.axis_index("i")
        a = jnp.bitwise_xor(p, 1)
        b = 3 - p

        barrier_sem = pltpu.get_barrier_semaphore()
        for nbr in [a, b]:
            pl.semaphore_signal(
                barrier_sem, inc=1,
                device_id=(nbr,), device_id_type=pl.DeviceIdType.MESH,
            )

        cp_dy = pltpu.make_async_copy(dy_hbm, dy_v, copy_sems.at[0])
        cp_w = pltpu.make_async_copy(w_hbm, w_v, copy_sems.at[1])
        cp_dy.start()
        cp_w.start()

        p_lt2 = p < 2
        p_even = (p % 2) == 0
        p_03 = (p == 0) | (p == 3)

        k1 = jnp.where(p_03, 0, 128)
        o2 = jnp.where(p_lt2, 0, 64)
        kB = jnp.where(p_lt2, 0, 128)
        oB = jnp.where(p_even, 0, 64)

        cp_dy.wait()
        cp_w.wait()

        def gemm_cols(c_off, width):
            out_ref[:, pl.ds(c_off, width)] = lax.dot_general(
                dy_v[...], w_v[pl.ds(c_off, width), :],
                dimension_numbers=(((1,), (1,)), ((), ())),
                preferred_element_type=jnp.float32,
            )

        def sem_idx(step, bf, ci):
            return (step * 2 + bf) * NC + ci

        def exch(step, bf, partner, row_off, nrows, c_off, ci, dst):
            r = pltpu.make_async_remote_copy(
                src_ref=out_ref.at[pl.ds(row_off, nrows), pl.ds(c_off, CW)],
                dst_ref=dst,
                send_sem=send_sems.at[sem_idx(step, bf, ci)],
                recv_sem=recv_sems.at[sem_idx(step, bf, ci)],
                device_id=(partner,),
                device_id_type=pl.DeviceIdType.MESH,
            )
            r.start()
            return r

        s1 = {}
        gemm_cols(0, 256)
        pl.semaphore_wait(barrier_sem, 2)
        for c, ci in COLS:
            if c >= 256:
                continue
            s1["B1", ci] = exch(0, 0, a, 128 - k1, 128, c, ci,
                                comm1.at[0, :, pl.ds(c, CW)])
            s1["B2", ci] = exch(0, 1, b, 384 - kB, 128, c, ci,
                                comm1.at[1, :, pl.ds(c, CW)])
        gemm_cols(256, 256)
        for c, ci in COLS:
            if c < 256:
                continue
            s1["B1", ci] = exch(0, 0, a, 128 - k1, 128, c, ci,
                                comm1.at[0, :, pl.ds(c, CW)])
            s1["B2", ci] = exch(0, 1, b, 384 - kB, 128, c, ci,
                                comm1.at[1, :, pl.ds(c, CW)])

        s2 = {}
        for c, ci in COLS:
            s1["B1", ci].wait_recv()
            out_ref[pl.ds(k1 + 64 - o2, 64), pl.ds(c, CW)] += (
                comm1[0, pl.ds(64 - o2, 64), pl.ds(c, CW)])
            s2["B1", ci] = exch(1, 0, b, k1 + 64 - o2, 64, c, ci,
                                comm2.at[0, :, pl.ds(c, CW)])
            s1["B2", ci].wait_recv()
            out_ref[pl.ds(256 + kB + 64 - oB, 64), pl.ds(c, CW)] += (
                comm1[1, pl.ds(64 - oB, 64), pl.ds(c, CW)])
            s2["B2", ci] = exch(1, 1, a, 256 + kB + 64 - oB, 64, c, ci,
                                comm2.at[1, :, pl.ds(c, CW)])

        out_ref[pl.ds(k1 + o2, 64), :] += comm1[0, pl.ds(o2, 64), :]
        out_ref[pl.ds(256 + kB + oB, 64), :] += comm1[1, pl.ds(oB, 64), :]

        s3 = {}
        s4a = {}
        for c, ci in COLS:
            s2["B1", ci].wait_recv()
            out_ref[pl.ds(k1 + o2, 64), pl.ds(c, CW)] += (
                comm2[0, :, pl.ds(c, CW)])
            s3["B1", ci] = exch(2, 0, b, k1 + o2, 64, c, ci,
                                out_ref.at[pl.ds(k1 + o2, 64), pl.ds(c, CW)])
            s4a["B1", ci] = exch(3, 0, a, k1 + o2, 64, c, ci,
                                 out_ref.at[pl.ds(k1 + o2, 64), pl.ds(c, CW)])
            s2["B2", ci].wait_recv()
            out_ref[pl.ds(256 + kB + oB, 64), pl.ds(c, CW)] += (
                comm2[1, :, pl.ds(c, CW)])
            s3["B2", ci] = exch(
                2, 1, a, 256 + kB + oB, 64, c, ci,
                out_ref.at[pl.ds(256 + kB + oB, 64), pl.ds(c, CW)])
            s4a["B2", ci] = exch(
                3, 1, b, 256 + kB + oB, 64, c, ci,
                out_ref.at[pl.ds(256 + kB + oB, 64), pl.ds(c, CW)])

        s4b = {}
        for c, ci in COLS:
            s3["B1", ci].wait_recv()
            s4b["B1", ci] = exch(
                4, 0, a, k1 + 64 - o2, 64, c, ci,
                out_ref.at[pl.ds(k1 + 64 - o2, 64), pl.ds(c, CW)])
            s3["B2", ci].wait_recv()
            s4b["B2", ci] = exch(
                4, 1, b, 256 + kB + 64 - oB, 64, c, ci,
                out_ref.at[pl.ds(256 + kB + 64 - oB, 64), pl.ds(c, CW)])

        for grp in (s4a, s4b):
            for r in grp.values():
                r.wait_recv()
        for grp in (s1, s2, s3, s4a, s4b):
            for r in grp.values():
                r.wait_send()

    n_sems = 5 * 2 * NC
    return pl.pallas_call(
        body,
        out_shape=jax.ShapeDtypeStruct((m, d), jnp.float32),
        in_specs=[
            pl.BlockSpec(memory_space=pltpu.MemorySpace.HBM),
            pl.BlockSpec(memory_space=pltpu.MemorySpace.HBM),
        ],
        out_specs=pl.BlockSpec(memory_space=pltpu.VMEM),
        scratch_shapes=[
            pltpu.VMEM((m, k), jnp.float32),
            pltpu.VMEM((d, k), jnp.float32),
            pltpu.VMEM((2, 128, d), jnp.float32),
            pltpu.VMEM((2, 64, d), jnp.float32),
            pltpu.SemaphoreType.DMA((2,)),
            pltpu.SemaphoreType.DMA((n_sems,)),
            pltpu.SemaphoreType.DMA((n_sems,)),
        ],
        compiler_params=pltpu.CompilerParams(collective_id=0),
    )(dy, W)
